# baseline (speedup 1.0000x reference)
import jax
import jax.numpy as jnp
from jax import lax
from jax.experimental import pallas as pl
from jax.experimental.pallas import tpu as pltpu
from jax.experimental.pallas import tpu_sc as plsc

B = 4
C = 19
H = 512
W = 512
HW = H * W
P = B * HW
NBC = B * C

NB = 256
L = 16
NC = 2
NS = 16
NW = NC * NS
PPW = HW // NW
HW4 = HW // 4
PPW4 = PPW // 4
CNB = C * NB
HIST = CNB * L
WGRP = PPW4 // L
FGW = 4096
NPBUF = 4


def _sc_hist_body(p_hbm, l_hbm, out_hbm, hist_v, fold_v, lab_v,
                  p0_v, p1_v, p2_v, p3_v,
                  sem_p0, sem_p1, sem_p2, sem_p3, sem_lab):
    wid = lax.axis_index("s") * NC + lax.axis_index("c")
    lane = lax.iota(jnp.int32, L)
    zeros = jnp.zeros((L,), jnp.float32)
    pbufs = (p0_v, p1_v, p2_v, p3_v)
    psems = (sem_p0, sem_p1, sem_p2, sem_p3)

    for b in range(B):
        pltpu.async_copy(
            l_hbm.at[pl.ds(b * HW4 + wid * PPW4, PPW4)],
            lab_v.at[pl.ds(b * PPW4, PPW4)],
            sem_lab,
        )

    def zero_body(i, _):
        for u in range(8):
            hist_v[pl.ds((i * 8 + u) * L, L)] = zeros
        return _

    lax.fori_loop(0, HIST // (L * 8), zero_body, None)

    for b in range(B):
        pltpu.make_async_copy(
            l_hbm.at[pl.ds(b * HW4 + wid * PPW4, PPW4)],
            lab_v.at[pl.ds(b * PPW4, PPW4)],
            sem_lab,
        ).wait()

    def start_p(bc, p_ref, sem):
        pltpu.async_copy(p_hbm.at[pl.ds(bc * HW4 + wid * PPW4, PPW4)], p_ref, sem)

    def wait_p(p_ref, sem):
        pltpu.make_async_copy(p_hbm.at[pl.ds(wid * PPW4, PPW4)], p_ref, sem).wait()

    def process(bc, p_ref):
        c = lax.rem(bc, C)
        loff = lax.div(bc, C) * PPW4
        cbase = c * (NB * L)

        def word_body(i, _):
            pw = p_ref[pl.ds(i * L, L)]
            lw = lab_v[pl.ds(loff + i * L, L)]
            for s in range(4):
                q = jnp.bitwise_and(lax.shift_right_logical(pw, 8 * s), 255)
                lq = jnp.bitwise_and(lax.shift_right_logical(lw, 8 * s), 255)
                fg = lq == c
                bidx = jnp.where(fg, (NB - 1) - q, q)
                nidx = cbase + bidx * L + lane
                val = jnp.where(fg, float(1 + FGW), 1.0)
                plsc.addupdate_scatter(hist_v, [nidx], val)
            return _

        lax.fori_loop(0, WGRP, word_body, None)

    for u in range(NPBUF):
        start_p(u, pbufs[u], psems[u])

    def outer(j, _):
        for u in range(NPBUF):
            bc = j * NPBUF + u
            wait_p(pbufs[u], psems[u])
            process(bc, pbufs[u])

            @pl.when(bc + NPBUF < NBC)
            def _():
                start_p(bc + NPBUF, pbufs[u], psems[u])

        return _

    lax.fori_loop(0, NBC // NPBUF, outer, None)

    def fold_body(j0, _):
        base = (j0 * L + lane) * L
        nacc = jnp.zeros((L,), jnp.int32)
        kacc = jnp.zeros((L,), jnp.int32)
        for t in range(L):
            sub = jnp.bitwise_and(lane + t, L - 1)
            v0 = plsc.load_gather(hist_v, [base + sub]).astype(jnp.int32)
            nacc = nacc + jnp.bitwise_and(v0, FGW - 1)
            kacc = kacc + (v0 >> 12)
        fold_v[pl.ds(j0 * L, L)] = nacc.astype(jnp.float32)
        fold_v[pl.ds(CNB + j0 * L, L)] = kacc.astype(jnp.float32)
        return _

    lax.fori_loop(0, CNB // L, fold_body, None)
    pltpu.sync_copy(fold_v, out_hbm.at[wid])


def _sc_histograms(pw, lw):
    mesh = plsc.VectorSubcoreMesh(
        core_axis_name="c", subcore_axis_name="s", num_cores=NC, num_subcores=NS
    )
    return pl.kernel(
        _sc_hist_body,
        out_type=jax.ShapeDtypeStruct((NW, 2 * CNB), jnp.float32),
        mesh=mesh,
        compiler_params=pltpu.CompilerParams(needs_layout_passes=False),
        scratch_types=[
            pltpu.VMEM((HIST,), jnp.float32),
            pltpu.VMEM((2 * CNB,), jnp.float32),
            pltpu.VMEM((B * PPW4,), jnp.int32),
            pltpu.VMEM((PPW4,), jnp.int32),
            pltpu.VMEM((PPW4,), jnp.int32),
            pltpu.VMEM((PPW4,), jnp.int32),
            pltpu.VMEM((PPW4,), jnp.int32),
            pltpu.SemaphoreType.DMA,
            pltpu.SemaphoreType.DMA,
            pltpu.SemaphoreType.DMA,
            pltpu.SemaphoreType.DMA,
            pltpu.SemaphoreType.DMA,
        ],
    )(pw, lw)


def _tc_loss_body(h_ref, o_ref):
    h = h_ref[...]
    s = jnp.sum(h, axis=0)
    n = s[:C, :]
    k = s[C:, :]
    row = lax.broadcasted_iota(jnp.int32, (NB, NB), 0)
    col = lax.broadcasted_iota(jnp.int32, (NB, NB), 1)
    tri = (row >= col).astype(jnp.float32)
    sn = jnp.dot(n, tri, preferred_element_type=jnp.float32)
    sk = jnp.dot(k, tri, preferred_element_type=jnp.float32)
    g = sk[:, 0:1]
    denom = jnp.maximum(g + sn - sk, 1.0)
    jac = 1.0 - (g - sk) / denom
    cw = lax.broadcasted_iota(jnp.int32, (1, NB), 1)
    wgt = jnp.where(cw == 0, 0.5, 1.0)
    losses = jnp.sum(jac * wgt, axis=1, keepdims=True) * (1.0 / NB)
    pres = (g > 0.0).astype(jnp.float32)
    loss = jnp.sum(losses * pres) / jnp.maximum(jnp.sum(pres), 1.0)
    o_ref[...] = jnp.reshape(loss, (1, 1))


def _tc_loss(h3):
    return pl.pallas_call(
        _tc_loss_body,
        out_shape=jax.ShapeDtypeStruct((1, 1), jnp.float32),
    )(h3)


@jax.jit
def kernel(probas, labels):
    p8 = (probas.reshape(-1) * float(NB)).astype(jnp.uint8)
    pw = lax.bitcast_convert_type(p8.reshape(-1, 4), jnp.int32)
    l8 = labels.reshape(-1).astype(jnp.uint8)
    lw = lax.bitcast_convert_type(l8.reshape(-1, 4), jnp.int32)
    hist = _sc_histograms(pw, lw)
    h3 = hist.reshape(NW, 2 * C, NB)
    out = _tc_loss(h3)
    return out[0, 0]


# speedup vs baseline: 5.7206x; 5.7206x over previous
import jax
import jax.numpy as jnp
from jax import lax
from jax.experimental import pallas as pl
from jax.experimental.pallas import tpu as pltpu
from jax.experimental.pallas import tpu_sc as plsc

B = 4
C = 19
H = 512
W = 512
HW = H * W
P = B * HW
NBC = B * C

NB = 128
L = 16
NC = 2
NS = 16
NW = NC * NS
PPW = HW // NW
VGRP = PPW // L
CNB = C * NB
HIST = CNB * L
FGW = 4096
NPBUF = 4


def _sc_hist_body(p_hbm, l_hbm, out_hbm, hist_v, fold_v, lab_v,
                  p0_v, p1_v, p2_v, p3_v,
                  sem_p0, sem_p1, sem_p2, sem_p3, sem_lab):
    wid = lax.axis_index("s") * NC + lax.axis_index("c")
    lane = lax.iota(jnp.int32, L)
    zeros = jnp.zeros((L,), jnp.float32)
    pbufs = (p0_v, p1_v, p2_v, p3_v)
    psems = (sem_p0, sem_p1, sem_p2, sem_p3)

    for b in range(B):
        pltpu.async_copy(
            l_hbm.at[pl.ds(b * HW + wid * PPW, PPW)],
            lab_v.at[pl.ds(b * PPW, PPW)],
            sem_lab,
        )

    def zero_body(i, _):
        for u in range(8):
            hist_v[pl.ds((i * 8 + u) * L, L)] = zeros
        return _

    lax.fori_loop(0, HIST // (L * 8), zero_body, None)

    for b in range(B):
        pltpu.make_async_copy(
            l_hbm.at[pl.ds(b * HW + wid * PPW, PPW)],
            lab_v.at[pl.ds(b * PPW, PPW)],
            sem_lab,
        ).wait()

    def start_p(bc, p_ref, sem):
        pltpu.async_copy(p_hbm.at[pl.ds(bc * HW + wid * PPW, PPW)], p_ref, sem)

    def wait_p(p_ref, sem):
        pltpu.make_async_copy(p_hbm.at[pl.ds(wid * PPW, PPW)], p_ref, sem).wait()

    def process(bc, p_ref):
        c = lax.rem(bc, C)
        loff = lax.div(bc, C) * PPW
        cbase = c * (NB * L)

        def vec_body(i, _):
            p = p_ref[pl.ds(i * L, L)]
            lb = lab_v[pl.ds(loff + i * L, L)]
            fg = lb == c
            e = jnp.where(fg, 1.0 - p, p)
            q = jnp.minimum((e * float(NB)).astype(jnp.int32), NB - 1)
            nidx = cbase + q * L + lane
            val = jnp.where(fg, float(1 + FGW), 1.0)
            plsc.addupdate_scatter(hist_v, [nidx], val)
            return _

        lax.fori_loop(0, VGRP, vec_body, None)

    for u in range(NPBUF):
        start_p(u, pbufs[u], psems[u])

    def outer(j, _):
        for u in range(NPBUF):
            bc = j * NPBUF + u
            wait_p(pbufs[u], psems[u])
            process(bc, pbufs[u])

            @pl.when(bc + NPBUF < NBC)
            def _():
                start_p(bc + NPBUF, pbufs[u], psems[u])

        return _

    lax.fori_loop(0, NBC // NPBUF, outer, None)

    def fold_body(j0, _):
        base = (j0 * L + lane) * L
        nacc = jnp.zeros((L,), jnp.int32)
        kacc = jnp.zeros((L,), jnp.int32)
        for t in range(L):
            sub = jnp.bitwise_and(lane + t, L - 1)
            v0 = plsc.load_gather(hist_v, [base + sub]).astype(jnp.int32)
            nacc = nacc + jnp.bitwise_and(v0, FGW - 1)
            kacc = kacc + (v0 >> 12)
        fold_v[pl.ds(j0 * L, L)] = nacc.astype(jnp.float32)
        fold_v[pl.ds(CNB + j0 * L, L)] = kacc.astype(jnp.float32)
        return _

    lax.fori_loop(0, CNB // L, fold_body, None)
    pltpu.sync_copy(fold_v, out_hbm.at[wid])


def _sc_histograms(pw, lw):
    mesh = plsc.VectorSubcoreMesh(
        core_axis_name="c", subcore_axis_name="s", num_cores=NC, num_subcores=NS
    )
    return pl.kernel(
        _sc_hist_body,
        out_type=jax.ShapeDtypeStruct((NW, 2 * CNB), jnp.float32),
        mesh=mesh,
        compiler_params=pltpu.CompilerParams(needs_layout_passes=False),
        scratch_types=[
            pltpu.VMEM((HIST,), jnp.float32),
            pltpu.VMEM((2 * CNB,), jnp.float32),
            pltpu.VMEM((B * PPW,), jnp.int32),
            pltpu.VMEM((PPW,), jnp.float32),
            pltpu.VMEM((PPW,), jnp.float32),
            pltpu.VMEM((PPW,), jnp.float32),
            pltpu.VMEM((PPW,), jnp.float32),
            pltpu.SemaphoreType.DMA,
            pltpu.SemaphoreType.DMA,
            pltpu.SemaphoreType.DMA,
            pltpu.SemaphoreType.DMA,
            pltpu.SemaphoreType.DMA,
        ],
    )(pw, lw)


def _tc_loss_body(h_ref, o_ref):
    h = h_ref[...]
    s = jnp.sum(h, axis=0)
    n = s[:C, :]
    k = s[C:, :]
    row = lax.broadcasted_iota(jnp.int32, (NB, NB), 0)
    col = lax.broadcasted_iota(jnp.int32, (NB, NB), 1)
    tri = (row >= col).astype(jnp.float32)
    sn = jnp.dot(n, tri, preferred_element_type=jnp.float32)
    sk = jnp.dot(k, tri, preferred_element_type=jnp.float32)
    g = sk[:, 0:1]
    denom = jnp.maximum(g + sn - sk, 1.0)
    jac = 1.0 - (g - sk) / denom
    cw = lax.broadcasted_iota(jnp.int32, (1, NB), 1)
    wgt = jnp.where(cw == 0, 0.5, 1.0)
    losses = jnp.sum(jac * wgt, axis=1, keepdims=True) * (1.0 / NB)
    pres = (g > 0.0).astype(jnp.float32)
    loss = jnp.sum(losses * pres) / jnp.maximum(jnp.sum(pres), 1.0)
    o_ref[...] = jnp.reshape(loss, (1, 1))


def _tc_loss(h3):
    return pl.pallas_call(
        _tc_loss_body,
        out_shape=jax.ShapeDtypeStruct((1, 1), jnp.float32),
    )(h3)


@jax.jit
def kernel(probas, labels):
    pw = probas.reshape(-1)
    lw = labels.reshape(-1).astype(jnp.int32)
    hist = _sc_histograms(pw, lw)
    h3 = hist.reshape(NW, 2 * C, NB)
    out = _tc_loss(h3)
    return out[0, 0]


# speedup vs baseline: 6.0904x; 1.0646x over previous
import jax
import jax.numpy as jnp
from jax import lax
from jax.experimental import pallas as pl
from jax.experimental.pallas import tpu as pltpu
from jax.experimental.pallas import tpu_sc as plsc

B = 4
C = 19
H = 512
W = 512
HW = H * W
P = B * HW
NBC = B * C

NB = 128
L = 16
NC = 2
NS = 16
NW = NC * NS
PPW = HW // NW
VGRP = PPW // L
CNB = C * NB
HIST = CNB * L
FGW = 4096
NPBUF = 4


def _sc_hist_body(p_hbm, l_hbm, out_hbm, hist_v, fold_v, lab_v,
                  p0_v, p1_v, p2_v, p3_v,
                  sem_p0, sem_p1, sem_p2, sem_p3, sem_lab):
    wid = lax.axis_index("s") * NC + lax.axis_index("c")
    lane = lax.iota(jnp.int32, L)
    zeros = jnp.zeros((L,), jnp.float32)
    pbufs = (p0_v, p1_v, p2_v, p3_v)
    psems = (sem_p0, sem_p1, sem_p2, sem_p3)

    for b in range(B):
        pltpu.async_copy(
            l_hbm.at[pl.ds(b * HW + wid * PPW, PPW)],
            lab_v.at[pl.ds(b * PPW, PPW)],
            sem_lab,
        )

    def zero_body(i, _):
        for u in range(8):
            hist_v[pl.ds((i * 8 + u) * L, L)] = zeros
        return _

    lax.fori_loop(0, HIST // (L * 8), zero_body, None)

    for b in range(B):
        pltpu.make_async_copy(
            l_hbm.at[pl.ds(b * HW + wid * PPW, PPW)],
            lab_v.at[pl.ds(b * PPW, PPW)],
            sem_lab,
        ).wait()

    def start_p(bc, p_ref, sem):
        pltpu.async_copy(p_hbm.at[pl.ds(bc * HW + wid * PPW, PPW)], p_ref, sem)

    def wait_p(p_ref, sem):
        pltpu.make_async_copy(p_hbm.at[pl.ds(wid * PPW, PPW)], p_ref, sem).wait()

    def process(bc, p_ref):
        c = lax.rem(bc, C)
        loff = lax.div(bc, C) * PPW
        cbase = c * (NB * L)

        def vec_body(i, _):
            for u in range(4):
                p = p_ref[pl.ds((i * 4 + u) * L, L)]
                lb = lab_v[pl.ds(loff + (i * 4 + u) * L, L)]
                fg = lb == c
                e = jnp.where(fg, 1.0 - p, p)
                q = jnp.minimum((e * float(NB)).astype(jnp.int32), NB - 1)
                nidx = cbase + q * L + lane
                val = jnp.where(fg, float(1 + FGW), 1.0)
                plsc.addupdate_scatter(hist_v, [nidx], val)
            return _

        lax.fori_loop(0, VGRP // 4, vec_body, None)

    for u in range(NPBUF):
        start_p(u, pbufs[u], psems[u])

    def outer(j, _):
        for u in range(NPBUF):
            bc = j * NPBUF + u
            wait_p(pbufs[u], psems[u])
            process(bc, pbufs[u])

            @pl.when(bc + NPBUF < NBC)
            def _():
                start_p(bc + NPBUF, pbufs[u], psems[u])

        return _

    lax.fori_loop(0, NBC // NPBUF, outer, None)

    def fold_body(j0, _):
        base = (j0 * L + lane) * L
        nacc = jnp.zeros((L,), jnp.int32)
        kacc = jnp.zeros((L,), jnp.int32)
        for t in range(L):
            sub = jnp.bitwise_and(lane + t, L - 1)
            v0 = plsc.load_gather(hist_v, [base + sub]).astype(jnp.int32)
            nacc = nacc + jnp.bitwise_and(v0, FGW - 1)
            kacc = kacc + (v0 >> 12)
        fold_v[pl.ds(j0 * L, L)] = nacc.astype(jnp.float32)
        fold_v[pl.ds(CNB + j0 * L, L)] = kacc.astype(jnp.float32)
        return _

    lax.fori_loop(0, CNB // L, fold_body, None)
    pltpu.sync_copy(fold_v, out_hbm.at[wid])


def _sc_histograms(pw, lw):
    mesh = plsc.VectorSubcoreMesh(
        core_axis_name="c", subcore_axis_name="s", num_cores=NC, num_subcores=NS
    )
    return pl.kernel(
        _sc_hist_body,
        out_type=jax.ShapeDtypeStruct((NW, 2 * CNB), jnp.float32),
        mesh=mesh,
        compiler_params=pltpu.CompilerParams(needs_layout_passes=False),
        scratch_types=[
            pltpu.VMEM((HIST,), jnp.float32),
            pltpu.VMEM((2 * CNB,), jnp.float32),
            pltpu.VMEM((B * PPW,), jnp.int32),
            pltpu.VMEM((PPW,), jnp.float32),
            pltpu.VMEM((PPW,), jnp.float32),
            pltpu.VMEM((PPW,), jnp.float32),
            pltpu.VMEM((PPW,), jnp.float32),
            pltpu.SemaphoreType.DMA,
            pltpu.SemaphoreType.DMA,
            pltpu.SemaphoreType.DMA,
            pltpu.SemaphoreType.DMA,
            pltpu.SemaphoreType.DMA,
        ],
    )(pw, lw)


def _tc_loss_body(h_ref, o_ref):
    h = h_ref[...]
    s = jnp.sum(h, axis=0)
    n = s[:C, :]
    k = s[C:, :]
    row = lax.broadcasted_iota(jnp.int32, (NB, NB), 0)
    col = lax.broadcasted_iota(jnp.int32, (NB, NB), 1)
    tri = (row >= col).astype(jnp.float32)
    sn = jnp.dot(n, tri, preferred_element_type=jnp.float32)
    sk = jnp.dot(k, tri, preferred_element_type=jnp.float32)
    g = sk[:, 0:1]
    denom = jnp.maximum(g + sn - sk, 1.0)
    jac = 1.0 - (g - sk) / denom
    cw = lax.broadcasted_iota(jnp.int32, (1, NB), 1)
    wgt = jnp.where(cw == 0, 0.5, 1.0)
    losses = jnp.sum(jac * wgt, axis=1, keepdims=True) * (1.0 / NB)
    pres = (g > 0.0).astype(jnp.float32)
    loss = jnp.sum(losses * pres) / jnp.maximum(jnp.sum(pres), 1.0)
    o_ref[...] = jnp.reshape(loss, (1, 1))


def _tc_loss(h3):
    return pl.pallas_call(
        _tc_loss_body,
        out_shape=jax.ShapeDtypeStruct((1, 1), jnp.float32),
    )(h3)


@jax.jit
def kernel(probas, labels):
    pw = probas.reshape(-1)
    lw = labels.reshape(-1).astype(jnp.int32)
    hist = _sc_histograms(pw, lw)
    h3 = hist.reshape(NW, 2 * C, NB)
    out = _tc_loss(h3)
    return out[0, 0]
